# Initial kernel scaffold; baseline (speedup 1.0000x reference)
import jax
import jax.numpy as jnp
from jax.experimental import pallas as pl


def kernel(danceability, energy, loudness, speechiness, acousticness, instrumentalness, liveness, valence, tempo, table_danceability, table_energy, table_loudness, table_speechiness, table_acousticness, table_instrumentalness, table_liveness, table_valence, table_tempo, id, table_id, genre, W, b):
    raise NotImplementedError("write your pallas kernel here")


# speedup vs baseline: 3.6272x; 3.6272x over previous
import functools

import jax
import jax.numpy as jnp
from jax import lax
from jax.experimental import pallas as pl
from jax.experimental.pallas import tpu as pltpu
from jax.experimental.pallas import tpu_sc as plsc

B = 16384
D = 128
NF = 9
VF = 100
KIN = 164
BG = 2048

NC = 2
NS = 16
NW = NC * NS
RPW = B // NW
CH = 64
NCHUNK = RPW // CH


def _tc_body(t_ref, w_ref, b_ref, genre_ref, p_ref, g_ref):
    @pl.when(pl.program_id(0) == 0)
    def _():
        for c in range(NF):
            p_ref[c] = lax.dot_general(
                t_ref[c], w_ref[:, 16 * c:16 * (c + 1)],
                (((1,), (1,)), ((), ())),
                preferred_element_type=jnp.float32,
                precision=lax.Precision.HIGHEST)
    wg = w_ref[:, 144:KIN]
    g_ref[...] = lax.dot_general(
        genre_ref[...], wg, (((1,), (1,)), ((), ())),
        preferred_element_type=jnp.float32,
        precision=lax.Precision.HIGHEST) + b_ref[...]


_tc_call = pl.pallas_call(
    _tc_body,
    grid=(B // BG,),
    in_specs=[
        pl.BlockSpec((NF, VF, 16), lambda i: (0, 0, 0)),
        pl.BlockSpec((D, KIN), lambda i: (0, 0)),
        pl.BlockSpec((1, D), lambda i: (0, 0)),
        pl.BlockSpec((BG, 20), lambda i: (i, 0)),
    ],
    out_specs=[
        pl.BlockSpec((NF, VF, D), lambda i: (0, 0, 0)),
        pl.BlockSpec((BG, D), lambda i: (i, 0)),
    ],
    out_shape=[
        jax.ShapeDtypeStruct((NF, VF, D), jnp.float32),
        jax.ShapeDtypeStruct((B, D), jnp.float32),
    ],
)


def _sc_body(p_hbm, tid_hbm, g_hbm, midx_hbm, id_hbm, out_hbm, ib, *rest):
    bufs = rest[:NF + 1]
    gbuf = rest[NF + 1]
    sem = rest[NF + 2]
    wid = lax.axis_index("s") * NC + lax.axis_index("c")
    base = wid * RPW
    for chunk in range(NCHUNK):
        cb = base + chunk * CH
        for t in range(NF):
            pltpu.sync_copy(midx_hbm.at[pl.ds(t * B + cb, CH)], ib.at[t])
        pltpu.sync_copy(id_hbm.at[pl.ds(cb, CH)], ib.at[NF])
        for t in range(1, NF):
            for k in range(CH // 16):
                s = pl.ds(16 * k, 16)
                ib[t, s] = ib[t, s] + VF * t
        copies = [pltpu.async_copy(p_hbm.at[ib.at[t]], bufs[t], sem)
                  for t in range(NF)]
        copies.append(pltpu.async_copy(tid_hbm.at[ib.at[NF]], bufs[NF], sem))
        copies.append(pltpu.async_copy(g_hbm.at[pl.ds(cb, CH)], gbuf, sem))
        for cpy in copies:
            cpy.wait()

        def row_body(r, carry):
            for j in range(D // 16):
                s = pl.ds(16 * j, 16)
                v = gbuf[r, s]
                for t in range(NF + 1):
                    v = v + bufs[t][r, s]
                gbuf[r, s] = v
            return carry
        lax.fori_loop(0, CH, row_body, 0)
        pltpu.sync_copy(gbuf, out_hbm.at[pl.ds(cb, CH)])


_sc_call = pl.kernel(
    _sc_body,
    mesh=plsc.VectorSubcoreMesh(core_axis_name="c", subcore_axis_name="s"),
    out_type=jax.ShapeDtypeStruct((B, D), jnp.float32),
    scratch_types=(
        [pltpu.VMEM((NF + 1, CH), jnp.int32)]
        + [pltpu.VMEM((CH, D), jnp.float32)] * (NF + 2)
        + [pltpu.SemaphoreType.DMA]
    ),
)


def kernel(danceability, energy, loudness, speechiness, acousticness,
           instrumentalness, liveness, valence, tempo,
           table_danceability, table_energy, table_loudness,
           table_speechiness, table_acousticness, table_instrumentalness,
           table_liveness, table_valence, table_tempo,
           id, table_id, genre, W, b):
    music_idx = [danceability, energy, loudness, speechiness, acousticness,
                 instrumentalness, liveness, valence, tempo]
    music_tables = [table_danceability, table_energy, table_loudness,
                    table_speechiness, table_acousticness,
                    table_instrumentalness, table_liveness, table_valence,
                    table_tempo]
    t_stack = jnp.stack(music_tables)
    midx = jnp.concatenate(
        [x.astype(jnp.int32) for x in music_idx])
    idv = id.astype(jnp.int32)
    p3, g = _tc_call(t_stack, W, b.reshape(1, D), genre)
    p = p3.reshape(NF * VF, D)
    return _sc_call(p, table_id, g, midx, idv)


# speedup vs baseline: 4.8956x; 1.3497x over previous
import functools

import jax
import jax.numpy as jnp
from jax import lax
from jax.experimental import pallas as pl
from jax.experimental.pallas import tpu as pltpu
from jax.experimental.pallas import tpu_sc as plsc

B = 16384
D = 128
NF = 9
VF = 100
KIN = 164
BG = 2048

NC = 2
NS = 16
NW = NC * NS
RPW = B // NW
CH = 32
NCHUNK = RPW // CH
NT = NF + 1
GPC = CH // 16


def _tc_body(t_ref, w_ref, b_ref, genre_ref, p_ref, g_ref):
    @pl.when(pl.program_id(0) == 0)
    def _():
        for c in range(NF):
            p_ref[c] = lax.dot_general(
                t_ref[c], w_ref[:, 16 * c:16 * (c + 1)],
                (((1,), (1,)), ((), ())),
                preferred_element_type=jnp.float32,
                precision=lax.Precision.HIGHEST)
    wg = w_ref[:, 144:KIN]
    g_ref[...] = lax.dot_general(
        genre_ref[...], wg, (((1,), (1,)), ((), ())),
        preferred_element_type=jnp.float32,
        precision=lax.Precision.HIGHEST) + b_ref[...]


_tc_call = pl.pallas_call(
    _tc_body,
    grid=(B // BG,),
    in_specs=[
        pl.BlockSpec((NF, VF, 16), lambda i: (0, 0, 0)),
        pl.BlockSpec((D, KIN), lambda i: (0, 0)),
        pl.BlockSpec((1, D), lambda i: (0, 0)),
        pl.BlockSpec((BG, 20), lambda i: (i, 0)),
    ],
    out_specs=[
        pl.BlockSpec((NF, VF, D), lambda i: (0, 0, 0)),
        pl.BlockSpec((BG, D), lambda i: (i, 0)),
    ],
    out_shape=[
        jax.ShapeDtypeStruct((NF, VF, D), jnp.float32),
        jax.ShapeDtypeStruct((B, D), jnp.float32),
    ],
)


def _sc_body(p_hbm, tid_hbm, g_hbm, midx_hbm, id_hbm, out_hbm,
             st, ib, *rest):
    bufs = [rest[par * NT:(par + 1) * NT] for par in range(2)]
    gb = rest[2 * NT:2 * NT + 2]
    gsem = rest[2 * NT + 2:2 * NT + 4]
    wsem = rest[2 * NT + 4:2 * NT + 6]
    wid = lax.axis_index("s") * NC + lax.axis_index("c")
    base = wid * RPW

    icopies = [pltpu.async_copy(midx_hbm.at[pl.ds(t * B + base, RPW)],
                                st.at[t], gsem[0]) for t in range(NF)]
    icopies.append(pltpu.async_copy(id_hbm.at[pl.ds(base, RPW)],
                                    st.at[NF], gsem[0]))
    for cpy in icopies:
        cpy.wait()
    for t in range(NT):
        for g in range(RPW // 16):
            v = st[t, pl.ds(16 * g, 16)]
            if 0 < t < NF:
                v = v + VF * t
            ib[t, g // GPC, pl.ds((g % GPC) * 16, 16)] = v

    def fire(k, par):
        cps = [pltpu.async_copy(p_hbm.at[ib.at[t, k]], bufs[par][t],
                                gsem[par]) for t in range(NF)]
        cps.append(pltpu.async_copy(tid_hbm.at[ib.at[NF, k]], bufs[par][NF],
                                    gsem[par]))
        cps.append(pltpu.async_copy(g_hbm.at[pl.ds(base + k * CH, CH)],
                                    gb[par], gsem[par]))
        return cps

    pend = {0: fire(0, 0)}
    wr = [None, None]
    for k in range(NCHUNK):
        par = k & 1
        npar = 1 - par
        if k + 1 < NCHUNK:
            if wr[npar] is not None:
                wr[npar].wait()
            pend[k + 1] = fire(k + 1, npar)
        for cpy in pend.pop(k):
            cpy.wait()

        def row_body(r, carry):
            for j in range(D // 16):
                s = pl.ds(16 * j, 16)
                v = gb[par][r, s]
                for t in range(NT):
                    v = v + bufs[par][t][r, s]
                gb[par][r, s] = v
            return carry
        lax.fori_loop(0, CH, row_body, 0)
        wr[par] = pltpu.async_copy(
            gb[par], out_hbm.at[pl.ds(base + k * CH, CH)], wsem[par])
    wr[0].wait()
    wr[1].wait()


_sc_call = pl.kernel(
    _sc_body,
    mesh=plsc.VectorSubcoreMesh(core_axis_name="c", subcore_axis_name="s"),
    out_type=jax.ShapeDtypeStruct((B, D), jnp.float32),
    scratch_types=(
        [pltpu.VMEM((NT, RPW), jnp.int32),
         pltpu.VMEM((NT, NCHUNK, CH), jnp.int32)]
        + [pltpu.VMEM((CH, D), jnp.float32)] * (2 * NT)
        + [pltpu.VMEM((CH, D), jnp.float32)] * 2
        + [pltpu.SemaphoreType.DMA] * 4
    ),
)


def kernel(danceability, energy, loudness, speechiness, acousticness,
           instrumentalness, liveness, valence, tempo,
           table_danceability, table_energy, table_loudness,
           table_speechiness, table_acousticness, table_instrumentalness,
           table_liveness, table_valence, table_tempo,
           id, table_id, genre, W, b):
    music_idx = [danceability, energy, loudness, speechiness, acousticness,
                 instrumentalness, liveness, valence, tempo]
    music_tables = [table_danceability, table_energy, table_loudness,
                    table_speechiness, table_acousticness,
                    table_instrumentalness, table_liveness, table_valence,
                    table_tempo]
    t_stack = jnp.stack(music_tables)
    midx = jnp.concatenate(
        [x.astype(jnp.int32) for x in music_idx])
    idv = id.astype(jnp.int32)
    p3, g = _tc_call(t_stack, W, b.reshape(1, D), genre)
    p = p3.reshape(NF * VF, D)
    return _sc_call(p, table_id, g, midx, idv)


# speedup vs baseline: 5.7003x; 1.1644x over previous
import jax
import jax.numpy as jnp
from jax import lax
from jax.experimental import pallas as pl
from jax.experimental.pallas import tpu as pltpu
from jax.experimental.pallas import tpu_sc as plsc

B = 16384
D = 128
NF = 9
VF = 100
PSTR = 104
PROWS = VF * PSTR
NP = 4
KIN = 164
BG = 2048

NC = 2
NS = 16
NW = NC * NS
RPW = B // NW
CH = 64
NCHUNK = RPW // CH
NTG = NP + 2
GPC = CH // 16


def _pair_body(tpair_ref, w_ref, pp_ref):
    pa = lax.dot_general(
        tpair_ref[0, 0], w_ref[0, :, 0:16],
        (((1,), (1,)), ((), ())), preferred_element_type=jnp.float32,
        precision=lax.Precision.HIGHEST)
    pb = lax.dot_general(
        tpair_ref[0, 1], w_ref[0, :, 16:32],
        (((1,), (1,)), ((), ())), preferred_element_type=jnp.float32,
        precision=lax.Precision.HIGHEST)
    for a in range(VF):
        pp_ref[PSTR * a:PSTR * a + VF, :] = pa[a:a + 1, :] + pb


_pair_call = pl.pallas_call(
    _pair_body,
    grid=(NP,),
    in_specs=[
        pl.BlockSpec((1, 2, VF, 16), lambda p: (p, 0, 0, 0)),
        pl.BlockSpec((1, D, 32), lambda p: (p, 0, 0)),
    ],
    out_specs=pl.BlockSpec((PROWS, D), lambda p: (p, 0)),
    out_shape=jax.ShapeDtypeStruct((NP * PROWS, D), jnp.float32),
)


def _tc_body(t8_ref, w_ref, b_ref, genre_ref, p8_ref, g_ref):
    @pl.when(pl.program_id(0) == 0)
    def _():
        p8_ref[...] = lax.dot_general(
            t8_ref[...], w_ref[:, 128:144],
            (((1,), (1,)), ((), ())), preferred_element_type=jnp.float32,
            precision=lax.Precision.HIGHEST)
    wg = w_ref[:, 144:KIN]
    g_ref[...] = lax.dot_general(
        genre_ref[...], wg, (((1,), (1,)), ((), ())),
        preferred_element_type=jnp.float32,
        precision=lax.Precision.HIGHEST) + b_ref[...]


_tc_call = pl.pallas_call(
    _tc_body,
    grid=(B // BG,),
    in_specs=[
        pl.BlockSpec((VF, 16), lambda i: (0, 0)),
        pl.BlockSpec((D, KIN), lambda i: (0, 0)),
        pl.BlockSpec((1, D), lambda i: (0, 0)),
        pl.BlockSpec((BG, 20), lambda i: (i, 0)),
    ],
    out_specs=[
        pl.BlockSpec((VF, D), lambda i: (0, 0)),
        pl.BlockSpec((BG, D), lambda i: (i, 0)),
    ],
    out_shape=[
        jax.ShapeDtypeStruct((VF, D), jnp.float32),
        jax.ShapeDtypeStruct((B, D), jnp.float32),
    ],
)


def _sc_body(pp_hbm, p8_hbm, tid_hbm, g_hbm,
             i0, i1, i2, i3, i4, i5, i6, i7, i8, idv_hbm,
             out_hbm, st, ib, *rest):
    idx_in = [i0, i1, i2, i3, i4, i5, i6, i7, i8, idv_hbm]
    bufs = [rest[par * NTG:(par + 1) * NTG] for par in range(2)]
    gb = rest[2 * NTG:2 * NTG + 2]
    gsem = rest[2 * NTG + 2:2 * NTG + 4]
    wsem = rest[2 * NTG + 4:2 * NTG + 6]
    wid = lax.axis_index("s") * NC + lax.axis_index("c")
    base = wid * RPW

    icopies = [pltpu.async_copy(idx_in[t].at[pl.ds(base, RPW)],
                                st.at[t], gsem[0]) for t in range(NF + 1)]
    for cpy in icopies:
        cpy.wait()
    for g in range(RPW // 16):
        k = g // GPC
        o = pl.ds((g % GPC) * 16, 16)
        s = pl.ds(16 * g, 16)
        for p in range(NP):
            ib[p, k, o] = (st[2 * p, s] * PSTR + st[2 * p + 1, s]
                           + p * PROWS)
        ib[NP, k, o] = st[8, s]
        ib[NP + 1, k, o] = st[9, s]

    def fire(k, par):
        cps = [pltpu.async_copy(pp_hbm.at[ib.at[p, k]], bufs[par][p],
                                gsem[par]) for p in range(NP)]
        cps.append(pltpu.async_copy(p8_hbm.at[ib.at[NP, k]], bufs[par][NP],
                                    gsem[par]))
        cps.append(pltpu.async_copy(tid_hbm.at[ib.at[NP + 1, k]],
                                    bufs[par][NP + 1], gsem[par]))
        cps.append(pltpu.async_copy(g_hbm.at[pl.ds(base + k * CH, CH)],
                                    gb[par], gsem[par]))
        return cps

    pend = {0: fire(0, 0)}
    wr = [None, None]
    for k in range(NCHUNK):
        par = k & 1
        npar = 1 - par
        if k + 1 < NCHUNK:
            if wr[npar] is not None:
                wr[npar].wait()
            pend[k + 1] = fire(k + 1, npar)
        for cpy in pend.pop(k):
            cpy.wait()

        def row_body(r, carry):
            for j in range(D // 16):
                s = pl.ds(16 * j, 16)
                v = gb[par][r, s]
                for t in range(NTG):
                    v = v + bufs[par][t][r, s]
                gb[par][r, s] = v
            return carry
        lax.fori_loop(0, CH, row_body, 0)
        wr[par] = pltpu.async_copy(
            gb[par], out_hbm.at[pl.ds(base + k * CH, CH)], wsem[par])
    wr[0].wait()
    wr[1].wait()


_sc_call = pl.kernel(
    _sc_body,
    mesh=plsc.VectorSubcoreMesh(core_axis_name="c", subcore_axis_name="s"),
    out_type=jax.ShapeDtypeStruct((B, D), jnp.float32),
    scratch_types=(
        [pltpu.VMEM((NF + 1, RPW), jnp.int32),
         pltpu.VMEM((NTG, NCHUNK, CH), jnp.int32)]
        + [pltpu.VMEM((CH, D), jnp.float32)] * (2 * NTG)
        + [pltpu.VMEM((CH, D), jnp.float32)] * 2
        + [pltpu.SemaphoreType.DMA] * 4
    ),
)


def kernel(danceability, energy, loudness, speechiness, acousticness,
           instrumentalness, liveness, valence, tempo,
           table_danceability, table_energy, table_loudness,
           table_speechiness, table_acousticness, table_instrumentalness,
           table_liveness, table_valence, table_tempo,
           id, table_id, genre, W, b):
    music_idx = [danceability, energy, loudness, speechiness, acousticness,
                 instrumentalness, liveness, valence, tempo]
    idx32 = [x.astype(jnp.int32) for x in music_idx]
    idv = id.astype(jnp.int32)
    tpairs = jnp.stack([table_danceability, table_energy, table_loudness,
                        table_speechiness, table_acousticness,
                        table_instrumentalness, table_liveness,
                        table_valence]).reshape(NP, 2, VF, 16)
    wpairs = W[:, :128].reshape(D, NP, 32).transpose(1, 0, 2)
    pp = _pair_call(tpairs, wpairs)
    p8, g = _tc_call(table_tempo, W, b.reshape(1, D), genre)
    return _sc_call(pp, p8, table_id, g, *idx32, idv)


# speedup vs baseline: 5.8491x; 1.0261x over previous
import jax
import jax.numpy as jnp
from jax import lax
from jax.experimental import pallas as pl
from jax.experimental.pallas import tpu as pltpu
from jax.experimental.pallas import tpu_sc as plsc

B = 16384
D = 128
NF = 9
VF = 100
PSTR = 104
PROWS = VF * PSTR
NP = 4
KIN = 164
BG = 2048

NC = 2
NS = 16
NW = NC * NS
RPW = B // NW
CH = 64
NCHUNK = RPW // CH
NTG = NP + 2
GPC = CH // 16


def _pair_body(tpair_ref, w_ref, pp_ref):
    pa = lax.dot_general(
        tpair_ref[0, 0], w_ref[0, :, 0:16],
        (((1,), (1,)), ((), ())), preferred_element_type=jnp.float32,
        precision=lax.Precision.HIGHEST)
    pb = lax.dot_general(
        tpair_ref[0, 1], w_ref[0, :, 16:32],
        (((1,), (1,)), ((), ())), preferred_element_type=jnp.float32,
        precision=lax.Precision.HIGHEST)
    for a in range(VF):
        pp_ref[PSTR * a:PSTR * a + VF, :] = pa[a:a + 1, :] + pb


_pair_call = pl.pallas_call(
    _pair_body,
    grid=(NP,),
    in_specs=[
        pl.BlockSpec((1, 2, VF, 16), lambda p: (p, 0, 0, 0)),
        pl.BlockSpec((1, D, 32), lambda p: (p, 0, 0)),
    ],
    out_specs=pl.BlockSpec((PROWS, D), lambda p: (p, 0)),
    out_shape=jax.ShapeDtypeStruct((NP * PROWS, D), jnp.float32),
)


def _tc_body(t8_ref, w_ref, b_ref, genre_ref, p8_ref, g_ref):
    @pl.when(pl.program_id(0) == 0)
    def _():
        p8_ref[...] = lax.dot_general(
            t8_ref[...], w_ref[:, 128:144],
            (((1,), (1,)), ((), ())), preferred_element_type=jnp.float32,
            precision=lax.Precision.HIGHEST)
    wg = w_ref[:, 144:KIN]
    g_ref[...] = lax.dot_general(
        genre_ref[...], wg, (((1,), (1,)), ((), ())),
        preferred_element_type=jnp.float32,
        precision=lax.Precision.HIGHEST) + b_ref[...]


_tc_call = pl.pallas_call(
    _tc_body,
    grid=(B // BG,),
    in_specs=[
        pl.BlockSpec((VF, 16), lambda i: (0, 0)),
        pl.BlockSpec((D, KIN), lambda i: (0, 0)),
        pl.BlockSpec((1, D), lambda i: (0, 0)),
        pl.BlockSpec((BG, 20), lambda i: (i, 0)),
    ],
    out_specs=[
        pl.BlockSpec((VF, D), lambda i: (0, 0)),
        pl.BlockSpec((BG, D), lambda i: (i, 0)),
    ],
    out_shape=[
        jax.ShapeDtypeStruct((VF, D), jnp.float32),
        jax.ShapeDtypeStruct((B, D), jnp.float32),
    ],
)


def _sc_body(pp_hbm, p8_hbm, tid_hbm, g_hbm, idx_hbm,
             out_hbm, st, ib, *rest):
    bufs = [rest[par * NTG:(par + 1) * NTG] for par in range(2)]
    gb = rest[2 * NTG:2 * NTG + 2]
    gsem = rest[2 * NTG + 2:2 * NTG + 4]
    wsem = rest[2 * NTG + 4:2 * NTG + 6]
    wid = lax.axis_index("s") * NC + lax.axis_index("c")
    base = wid * RPW

    icopies = [pltpu.async_copy(idx_hbm.at[t, pl.ds(base, RPW)],
                                st.at[t], gsem[0]) for t in range(NF + 1)]
    for cpy in icopies:
        cpy.wait()
    for g in range(RPW // 16):
        k = g // GPC
        o = pl.ds((g % GPC) * 16, 16)
        s = pl.ds(16 * g, 16)
        for p in range(NP):
            ib[p, k, o] = (st[2 * p, s] * PSTR + st[2 * p + 1, s]
                           + p * PROWS)
        ib[NP, k, o] = st[8, s]
        ib[NP + 1, k, o] = st[9, s]

    def fire(k, par):
        cps = [pltpu.async_copy(pp_hbm.at[ib.at[p, k]], bufs[par][p],
                                gsem[par]) for p in range(NP)]
        cps.append(pltpu.async_copy(p8_hbm.at[ib.at[NP, k]], bufs[par][NP],
                                    gsem[par]))
        cps.append(pltpu.async_copy(tid_hbm.at[ib.at[NP + 1, k]],
                                    bufs[par][NP + 1], gsem[par]))
        cps.append(pltpu.async_copy(g_hbm.at[pl.ds(base + k * CH, CH)],
                                    gb[par], gsem[par]))
        return cps

    pend = {0: fire(0, 0)}
    wr = [None, None]
    for k in range(NCHUNK):
        par = k & 1
        npar = 1 - par
        if k + 1 < NCHUNK:
            if wr[npar] is not None:
                wr[npar].wait()
            pend[k + 1] = fire(k + 1, npar)
        for cpy in pend.pop(k):
            cpy.wait()

        def row_body(r, carry):
            for j in range(D // 16):
                s = pl.ds(16 * j, 16)
                v = gb[par][r, s]
                for t in range(NTG):
                    v = v + bufs[par][t][r, s]
                gb[par][r, s] = v
            return carry
        lax.fori_loop(0, CH, row_body, 0)
        wr[par] = pltpu.async_copy(
            gb[par], out_hbm.at[pl.ds(base + k * CH, CH)], wsem[par])
    wr[0].wait()
    wr[1].wait()


_sc_call = pl.kernel(
    _sc_body,
    mesh=plsc.VectorSubcoreMesh(core_axis_name="c", subcore_axis_name="s"),
    out_type=jax.ShapeDtypeStruct((B, D), jnp.float32),
    scratch_types=(
        [pltpu.VMEM((NF + 1, RPW), jnp.int32),
         pltpu.VMEM((NTG, NCHUNK, CH), jnp.int32)]
        + [pltpu.VMEM((CH, D), jnp.float32)] * (2 * NTG)
        + [pltpu.VMEM((CH, D), jnp.float32)] * 2
        + [pltpu.SemaphoreType.DMA] * 4
    ),
)


def kernel(danceability, energy, loudness, speechiness, acousticness,
           instrumentalness, liveness, valence, tempo,
           table_danceability, table_energy, table_loudness,
           table_speechiness, table_acousticness, table_instrumentalness,
           table_liveness, table_valence, table_tempo,
           id, table_id, genre, W, b):
    idx = jnp.stack([danceability, energy, loudness, speechiness,
                     acousticness, instrumentalness, liveness, valence,
                     tempo, id]).astype(jnp.int32)
    tpairs = jnp.stack([table_danceability, table_energy, table_loudness,
                        table_speechiness, table_acousticness,
                        table_instrumentalness, table_liveness,
                        table_valence]).reshape(NP, 2, VF, 16)
    wpairs = W[:, :128].reshape(D, NP, 32).transpose(1, 0, 2)
    pp = _pair_call(tpairs, wpairs)
    p8, g = _tc_call(table_tempo, W, b.reshape(1, D), genre)
    return _sc_call(pp, p8, table_id, g, idx)
